# baseline (speedup 1.0000x reference)
import functools

import jax
import jax.numpy as jnp
from jax import lax
from jax.experimental import pallas as pl
from jax.experimental.pallas import tpu as pltpu
from jax.experimental.pallas import tpu_sc as plsc

N = 100000
C = 128
ROWS = 320
TROWS = 256
NC = 2
NS = 16
NW = NC * NS
L = 16
GROUPS = ROWS // L
TGROUPS = TROWS // L
FULL = 9
BUF = ROWS * C
TBUF = TROWS * C
IDXW = FULL * ROWS + TROWS
IB = 128
TILE_SLOTS = (0, 1, 3, 4, 6, 7)
SH_SLOTS = (2, 5, 8)
NSCAT = len(TILE_SLOTS + SH_SLOTS) * 3 + 2

_mesh = plsc.VectorSubcoreMesh(
    core_axis_name="c", subcore_axis_name="s", num_cores=NC, num_subcores=NS
)


@functools.partial(
    pl.kernel,
    out_type=jax.ShapeDtypeStruct((N * C,), jnp.float32),
    mesh=_mesh,
    compiler_params=pltpu.CompilerParams(needs_layout_passes=False),
    scratch_types=[
        pltpu.VMEM((IDXW,), jnp.int32),
        pltpu.VMEM((BUF,), jnp.float32),
        pltpu.VMEM((IB,), jnp.float32),
        pltpu.VMEM((3 * FULL + 2, IB), jnp.int32),
        pltpu.VMEM_SHARED((BUF,), jnp.float32),
        pltpu.SemaphoreType.DMA,
        (pltpu.SemaphoreType.DMA,) * 7,
        (pltpu.SemaphoreType.DMA,) * 3,
        pltpu.SemaphoreType.DMA,
    ],
)
def _onehot_sc(x_hbm, out_hbm, idx_v, zblk, ones_v, possh, sh,
               si, szt, szh, ssc):
    wid = lax.axis_index("s") * NC + lax.axis_index("c")
    sid = lax.axis_index("s")
    lane = lax.iota(jnp.int32, L)
    ones = jnp.ones((L,), jnp.float32)
    zeros = jnp.zeros((L,), jnp.float32)

    start = pl.multiple_of(((wid * N // NW) >> 3) << 3, 8)
    end = pl.multiple_of((((wid + 1) * N // NW) >> 3) << 3, 8)

    def row0(g):
        return start + g * ROWS

    pltpu.make_async_copy(
        x_hbm.at[pl.ds(start, FULL * ROWS)], idx_v.at[pl.ds(0, FULL * ROWS)], si
    ).start()
    pltpu.make_async_copy(
        x_hbm.at[pl.ds(pl.multiple_of(end - TROWS, 8), TROWS)],
        idx_v.at[pl.ds(FULL * ROWS, TROWS)],
        si,
    ).start()

    for u in range(IB // L):
        ones_v[pl.ds(u * L, L)] = ones

    def _zseg(t, _):
        base = t * ROWS
        for u in range(ROWS // L):
            zblk[pl.ds(base + u * L, L)] = zeros
        return 0

    lax.fori_loop(0, BUF // ROWS, _zseg, 0)

    @pl.when(sid == 0)
    def _():
        pltpu.sync_copy(zblk, sh)

    for k, g in enumerate(TILE_SLOTS):
        pltpu.make_async_copy(
            zblk,
            out_hbm.at[pl.ds(pl.multiple_of(row0(g) * C, 8), BUF)],
            szt[k],
        ).start()
    pltpu.make_async_copy(
        zblk.at[pl.ds(0, TBUF)],
        out_hbm.at[pl.ds(pl.multiple_of((end - TROWS) * C, 8), TBUF)],
        szt[6],
    ).start()

    plsc.subcore_barrier()

    for k, g in enumerate(SH_SLOTS):
        pltpu.make_async_copy(
            sh,
            out_hbm.at[pl.ds(pl.multiple_of(row0(g) * C, 8), BUF)],
            szh[k],
        ).start()

    pltpu.make_async_copy(
        x_hbm.at[pl.ds(0, FULL * ROWS)], idx_v.at[pl.ds(0, FULL * ROWS)], si
    ).wait()
    pltpu.make_async_copy(
        x_hbm.at[pl.ds(0, TROWS)], idx_v.at[pl.ds(FULL * ROWS, TROWS)], si
    ).wait()

    def _scatter_ones(g, base_row, ngroups, prow):
        def _grp(j, _):
            cols = idx_v[pl.ds(g * ROWS + j * L, L)]
            pos = (base_row + j * L + lane) * C + cols
            possh[prow + j // 8, pl.ds((j % 8) * L, L)] = pos

            @pl.when(j >= 2 * (IB // L))
            def _():
                possh[prow + 2, pl.ds((j % 8) * L + IB // 2, L)] = pos

            return 0

        lax.fori_loop(0, ngroups, _grp, 0)
        for r in range(3 if ngroups == GROUPS else 2):
            pltpu.make_async_copy(
                ones_v, out_hbm.at[possh.at[prow + r]], ssc
            ).start()

    for k, g in enumerate(TILE_SLOTS):
        pltpu.make_async_copy(
            zblk, out_hbm.at[pl.ds(0, BUF)], szt[k]
        ).wait()
        _scatter_ones(g, row0(g), GROUPS, 3 * g)
    for k, g in enumerate(SH_SLOTS):
        pltpu.make_async_copy(
            sh, out_hbm.at[pl.ds(0, BUF)], szh[k]
        ).wait()
        _scatter_ones(g, row0(g), GROUPS, 3 * g)
    pltpu.make_async_copy(
        zblk.at[pl.ds(0, TBUF)], out_hbm.at[pl.ds(0, TBUF)], szt[6]
    ).wait()
    _scatter_ones(FULL, end - TROWS, TGROUPS, 3 * FULL)

    for _ in range(NSCAT):
        pltpu.make_async_copy(ones_v, out_hbm.at[possh.at[0]], ssc).wait()


def kernel(x):
    return _onehot_sc(x).reshape(N, C)


# speedup vs baseline: 3.9258x; 3.9258x over previous
import functools

import jax
import jax.numpy as jnp
from jax import lax
from jax.experimental import pallas as pl
from jax.experimental.pallas import tpu as pltpu
from jax.experimental.pallas import tpu_sc as plsc

N = 100000
C = 128
ROWS = 448
TROWS = 448
NC = 2
NS = 16
NW = NC * NS
L = 16
GROUPS = ROWS // L
TGROUPS = TROWS // L
FULL = 6
BUF = ROWS * C
TBUF = TROWS * C
IDXW = FULL * ROWS + TROWS

_mesh = plsc.VectorSubcoreMesh(
    core_axis_name="c", subcore_axis_name="s", num_cores=NC, num_subcores=NS
)


@functools.partial(
    pl.kernel,
    out_type=jax.ShapeDtypeStruct((N * C,), jnp.float32),
    mesh=_mesh,
    compiler_params=pltpu.CompilerParams(needs_layout_passes=False),
    scratch_types=[
        pltpu.VMEM((IDXW,), jnp.int32),
        (pltpu.VMEM((BUF,), jnp.float32),) * 2,
        (pltpu.VMEM((ROWS,), jnp.int32),) * 2,
        pltpu.SemaphoreType.DMA,
        (pltpu.SemaphoreType.DMA,) * 2,
    ],
)
def _onehot_sc(x_hbm, out_hbm, idx_v, bufs, poss, si, sos):
    wid = lax.axis_index("s") * NC + lax.axis_index("c")
    lane = lax.iota(jnp.int32, L)
    ones = jnp.ones((L,), jnp.float32)
    zeros = jnp.zeros((L,), jnp.float32)

    start = pl.multiple_of(((wid * N // NW) >> 3) << 3, 8)
    end = pl.multiple_of((((wid + 1) * N // NW) >> 3) << 3, 8)

    pltpu.make_async_copy(
        x_hbm.at[pl.ds(start, FULL * ROWS)], idx_v.at[pl.ds(0, FULL * ROWS)], si
    ).start()
    pltpu.make_async_copy(
        x_hbm.at[pl.ds(pl.multiple_of(end - TROWS, 8), TROWS)],
        idx_v.at[pl.ds(FULL * ROWS, TROWS)],
        si,
    ).start()

    def _zero(b):
        def _seg(t, _):
            base = t * ROWS
            for u in range(ROWS // L):
                bufs[b][pl.ds(base + u * L, L)] = zeros
            return 0

        lax.fori_loop(0, BUF // ROWS, _seg, 0)

    _zero(0)

    pltpu.make_async_copy(
        x_hbm.at[pl.ds(0, FULL * ROWS)], idx_v.at[pl.ds(0, FULL * ROWS)], si
    ).wait()
    pltpu.make_async_copy(
        x_hbm.at[pl.ds(0, TROWS)], idx_v.at[pl.ds(FULL * ROWS, TROWS)], si
    ).wait()

    def _clean(b, ngroups):
        def _grp(j, _):
            p = poss[b][pl.ds(j * L, L)]
            plsc.store_scatter(bufs[b], [p], zeros)
            return 0

        lax.fori_loop(0, ngroups, _grp, 0)

    def _build(b, ibase, ngroups):
        def _grp(j, _):
            cols = idx_v[pl.ds(ibase + j * L, L)]
            pos = (j * L + lane) * C + cols
            plsc.store_scatter(bufs[b], [pos], ones)
            poss[b][pl.ds(j * L, L)] = pos
            return 0

        lax.fori_loop(0, ngroups, _grp, 0)

    def _send(b, i):
        pltpu.make_async_copy(
            bufs[b],
            out_hbm.at[pl.ds(pl.multiple_of((start + i * ROWS) * C, 8), BUF)],
            sos[b],
        ).start()

    def _slot(i, b):
        @pl.when(i >= 2)
        def _():
            pltpu.make_async_copy(
                bufs[b], out_hbm.at[pl.ds(0, BUF)], sos[b]
            ).wait()
            _clean(b, GROUPS)

        _build(b, i * ROWS, GROUPS)
        _send(b, i)

    _build(0, 0, GROUPS)
    _send(0, 0)
    _zero(1)

    for i in range(1, FULL):
        _slot(i, i % 2)

    pltpu.make_async_copy(bufs[0], out_hbm.at[pl.ds(0, BUF)], sos[0]).wait()
    _clean(0, GROUPS)
    _build(0, FULL * ROWS, TGROUPS)
    pltpu.make_async_copy(
        bufs[0],
        out_hbm.at[pl.ds(pl.multiple_of((end - TROWS) * C, 8), TBUF)],
        sos[0],
    ).start()

    pltpu.make_async_copy(bufs[1], out_hbm.at[pl.ds(0, BUF)], sos[1]).wait()
    pltpu.make_async_copy(bufs[0], out_hbm.at[pl.ds(0, TBUF)], sos[0]).wait()


def kernel(x):
    return _onehot_sc(x).reshape(N, C)


# speedup vs baseline: 4.0553x; 1.0330x over previous
import functools

import jax
import jax.numpy as jnp
from jax import lax
from jax.experimental import pallas as pl
from jax.experimental.pallas import tpu as pltpu
from jax.experimental.pallas import tpu_sc as plsc

N = 100000
C = 128
ROWS = 320
TROWS = 256
NC = 2
NS = 16
NW = NC * NS
L = 16
GROUPS = ROWS // L
TGROUPS = TROWS // L
FULL = 9
BUF = ROWS * C
TBUF = TROWS * C
IDXW = FULL * ROWS + TROWS

_mesh = plsc.VectorSubcoreMesh(
    core_axis_name="c", subcore_axis_name="s", num_cores=NC, num_subcores=NS
)


@functools.partial(
    pl.kernel,
    out_type=jax.ShapeDtypeStruct((N * C,), jnp.float32),
    mesh=_mesh,
    compiler_params=pltpu.CompilerParams(needs_layout_passes=False),
    scratch_types=[
        pltpu.VMEM((IDXW,), jnp.int32),
        (pltpu.VMEM((BUF,), jnp.float32),) * 2,
        (pltpu.VMEM((ROWS,), jnp.int32),) * 2,
        pltpu.SemaphoreType.DMA,
        (pltpu.SemaphoreType.DMA,) * 2,
    ],
)
def _onehot_sc(x_hbm, out_hbm, idx_v, bufs, poss, si, sos):
    wid = lax.axis_index("s") * NC + lax.axis_index("c")
    lane = lax.iota(jnp.int32, L)
    ones = jnp.ones((L,), jnp.float32)
    zeros = jnp.zeros((L,), jnp.float32)

    start = pl.multiple_of(((wid * N // NW) >> 3) << 3, 8)
    end = pl.multiple_of((((wid + 1) * N // NW) >> 3) << 3, 8)

    pltpu.make_async_copy(
        x_hbm.at[pl.ds(start, FULL * ROWS)], idx_v.at[pl.ds(0, FULL * ROWS)], si
    ).start()
    pltpu.make_async_copy(
        x_hbm.at[pl.ds(pl.multiple_of(end - TROWS, 8), TROWS)],
        idx_v.at[pl.ds(FULL * ROWS, TROWS)],
        si,
    ).start()

    def _zero(b):
        def _seg(t, _):
            base = t * ROWS
            for u in range(ROWS // L):
                bufs[b][pl.ds(base + u * L, L)] = zeros
            return 0

        lax.fori_loop(0, BUF // ROWS, _seg, 0)

    _zero(0)

    pltpu.make_async_copy(
        x_hbm.at[pl.ds(0, FULL * ROWS)], idx_v.at[pl.ds(0, FULL * ROWS)], si
    ).wait()
    pltpu.make_async_copy(
        x_hbm.at[pl.ds(0, TROWS)], idx_v.at[pl.ds(FULL * ROWS, TROWS)], si
    ).wait()

    def _clean(b, ngroups):
        def _grp(j, _):
            p = poss[b][pl.ds(j * L, L)]
            plsc.store_scatter(bufs[b], [p], zeros)
            return 0

        lax.fori_loop(0, ngroups, _grp, 0)

    def _build(b, ibase, ngroups):
        def _grp(j, _):
            cols = idx_v[pl.ds(ibase + j * L, L)]
            pos = (j * L + lane) * C + cols
            plsc.store_scatter(bufs[b], [pos], ones)
            poss[b][pl.ds(j * L, L)] = pos
            return 0

        lax.fori_loop(0, ngroups, _grp, 0)

    def _send(b, i):
        pltpu.make_async_copy(
            bufs[b],
            out_hbm.at[pl.ds(pl.multiple_of((start + i * ROWS) * C, 8), BUF)],
            sos[b],
        ).start()

    def _slot(i, b):
        @pl.when(i >= 2)
        def _():
            pltpu.make_async_copy(
                bufs[b], out_hbm.at[pl.ds(0, BUF)], sos[b]
            ).wait()
            _clean(b, GROUPS)

        _build(b, i * ROWS, GROUPS)
        _send(b, i)

    _build(0, 0, GROUPS)
    _send(0, 0)
    _zero(1)

    def _pair(t, _):
        _slot(2 * t + 1, 1)
        _slot(2 * t + 2, 0)
        return 0

    lax.fori_loop(0, (FULL - 1) // 2, _pair, 0)

    pltpu.make_async_copy(bufs[1], out_hbm.at[pl.ds(0, BUF)], sos[1]).wait()
    _clean(1, GROUPS)
    _build(1, FULL * ROWS, TGROUPS)
    pltpu.make_async_copy(
        bufs[1].at[pl.ds(0, TBUF)],
        out_hbm.at[pl.ds(pl.multiple_of((end - TROWS) * C, 8), TBUF)],
        sos[1],
    ).start()

    pltpu.make_async_copy(bufs[0], out_hbm.at[pl.ds(0, BUF)], sos[0]).wait()
    pltpu.make_async_copy(
        bufs[1].at[pl.ds(0, TBUF)], out_hbm.at[pl.ds(0, TBUF)], sos[1]
    ).wait()


def kernel(x):
    return _onehot_sc(x).reshape(N, C)
